# baseline (speedup 1.0000x reference)
import functools

import jax
import jax.numpy as jnp
from jax.experimental import pallas as pl
from jax.experimental.pallas import tpu as pltpu
from jax.experimental.pallas import tpu_sc as plsc

B = 4
S = 8192
D = 768
N = B * S
C = 128
NC = D // C
W = 128
NWIN = N // W

TC_ROWS = 2048

CHUNK_WINDOWS = (64, 192, 192, 64)
K = len(CHUNK_WINDOWS)


WR = 64
NWR = N // WR


def _gather_sc(idx_pad, table, n_windows):
    mesh = plsc.VectorSubcoreMesh(core_axis_name="c", subcore_axis_name="s")

    @functools.partial(
        pl.kernel,
        out_type=jax.ShapeDtypeStruct((n_windows * WR, D), jnp.float32),
        mesh=mesh,
    )
    def k(i_hbm, t_hbm, o_hbm):
        def body(i_vmem, o_vmem):
            pltpu.sync_copy(t_hbm.at[i_vmem.at[0, pl.ds(0, WR)]], o_vmem)

        pltpu.emit_pipeline(
            body,
            grid=(n_windows,),
            in_specs=[pl.BlockSpec((1, 128), lambda i: (i, 0))],
            out_specs=[pl.BlockSpec((WR, D), lambda i: (i, 0))],
            core_axis_name=("c", "s"),
            dimension_semantics=(pltpu.PARALLEL,),
        )(i_hbm, o_hbm)

    return k(idx_pad, table)


def _add_first(x2d, pe0, n_blocks):
    def body(x_ref, pe_ref, o_ref):
        o_ref[...] = x_ref[...] + pe_ref[...]

    return pl.pallas_call(
        body,
        out_shape=jax.ShapeDtypeStruct((N, D), jnp.float32),
        grid=(n_blocks,),
        in_specs=[
            pl.BlockSpec((TC_ROWS, D), lambda i: (i, 0)),
            pl.BlockSpec((TC_ROWS, D), lambda i: (i, 0)),
        ],
        out_specs=pl.BlockSpec((TC_ROWS, D), lambda i: (i, 0)),
    )(x2d, pe0)


def _add_chunk(prev, x2d, pe, off_blocks, n_blocks):
    def body(prev_ref, x_ref, pe_ref, o_ref):
        o_ref[...] = x_ref[...] + pe_ref[...]

    return pl.pallas_call(
        body,
        out_shape=jax.ShapeDtypeStruct((N, D), jnp.float32),
        grid=(n_blocks,),
        in_specs=[
            pl.BlockSpec(memory_space=pltpu.MemorySpace.HBM),
            pl.BlockSpec((TC_ROWS, D), lambda i: (i + off_blocks, 0)),
            pl.BlockSpec((TC_ROWS, D), lambda i: (i, 0)),
        ],
        out_specs=pl.BlockSpec((TC_ROWS, D), lambda i: (i + off_blocks, 0)),
        input_output_aliases={0: 0},
    )(prev, x2d, pe)


def kernel(x, position, pos_table):
    x2d = x.reshape(N, D)
    pos = position.reshape(NWR, WR).astype(jnp.int32)
    idx_pad = jnp.concatenate([pos, pos], axis=1)
    pe = []
    w0 = 0
    for nw in CHUNK_WINDOWS:
        pe.append(_gather_sc(idx_pad[w0:w0 + nw], pos_table, nw))
        w0 += nw
    out = _add_first(x2d, pe[0], CHUNK_WINDOWS[0] * WR // TC_ROWS)
    off = CHUNK_WINDOWS[0] * WR // TC_ROWS
    for k in range(1, K):
        nb = CHUNK_WINDOWS[k] * WR // TC_ROWS
        out = _add_chunk(out, x2d, pe[k], off, nb)
        off += nb
    return out.reshape(B, S, D)


# speedup vs baseline: 1.0121x; 1.0121x over previous
import functools

import jax
import jax.numpy as jnp
from jax.experimental import pallas as pl
from jax.experimental.pallas import tpu as pltpu
from jax.experimental.pallas import tpu_sc as plsc

B = 4
S = 8192
D = 768
N = B * S
C = 128
NC = D // C
W = 128
NWIN = N // W

TC_ROWS = 2048

CHUNK_WINDOWS = (256, 256)
K = len(CHUNK_WINDOWS)


WR = 64
NWR = N // WR


def _gather_sc(idx_pad, table, n_windows):
    mesh = plsc.VectorSubcoreMesh(core_axis_name="c", subcore_axis_name="s")

    @functools.partial(
        pl.kernel,
        out_type=jax.ShapeDtypeStruct((n_windows * WR, D), jnp.float32),
        mesh=mesh,
    )
    def k(i_hbm, t_hbm, o_hbm):
        def body(i_vmem, o_vmem):
            pltpu.sync_copy(t_hbm.at[i_vmem.at[0, pl.ds(0, WR)]], o_vmem)

        pltpu.emit_pipeline(
            body,
            grid=(n_windows,),
            in_specs=[pl.BlockSpec((1, 128), lambda i: (i, 0))],
            out_specs=[pl.BlockSpec((WR, D), lambda i: (i, 0))],
            core_axis_name=("c", "s"),
            dimension_semantics=(pltpu.PARALLEL,),
        )(i_hbm, o_hbm)

    return k(idx_pad, table)


def _add_first(x2d, pe0, n_blocks):
    def body(x_ref, pe_ref, o_ref):
        o_ref[...] = x_ref[...] + pe_ref[...]

    return pl.pallas_call(
        body,
        out_shape=jax.ShapeDtypeStruct((N, D), jnp.float32),
        grid=(n_blocks,),
        in_specs=[
            pl.BlockSpec((TC_ROWS, D), lambda i: (i, 0)),
            pl.BlockSpec((TC_ROWS, D), lambda i: (i, 0)),
        ],
        out_specs=pl.BlockSpec((TC_ROWS, D), lambda i: (i, 0)),
    )(x2d, pe0)


def _add_chunk(prev, x2d, pe, off_blocks, n_blocks):
    def body(prev_ref, x_ref, pe_ref, o_ref):
        o_ref[...] = x_ref[...] + pe_ref[...]

    return pl.pallas_call(
        body,
        out_shape=jax.ShapeDtypeStruct((N, D), jnp.float32),
        grid=(n_blocks,),
        in_specs=[
            pl.BlockSpec(memory_space=pltpu.MemorySpace.HBM),
            pl.BlockSpec((TC_ROWS, D), lambda i: (i + off_blocks, 0)),
            pl.BlockSpec((TC_ROWS, D), lambda i: (i, 0)),
        ],
        out_specs=pl.BlockSpec((TC_ROWS, D), lambda i: (i + off_blocks, 0)),
        input_output_aliases={0: 0},
    )(prev, x2d, pe)


def kernel(x, position, pos_table):
    x2d = x.reshape(N, D)
    pos = position.reshape(NWR, WR).astype(jnp.int32)
    idx_pad = jnp.concatenate([pos, pos], axis=1)
    pe = []
    w0 = 0
    for nw in CHUNK_WINDOWS:
        pe.append(_gather_sc(idx_pad[w0:w0 + nw], pos_table, nw))
        w0 += nw
    out = _add_first(x2d, pe[0], CHUNK_WINDOWS[0] * WR // TC_ROWS)
    off = CHUNK_WINDOWS[0] * WR // TC_ROWS
    for k in range(1, K):
        nb = CHUNK_WINDOWS[k] * WR // TC_ROWS
        out = _add_chunk(out, x2d, pe[k], off, nb)
        off += nb
    return out.reshape(B, S, D)
